# baseline (speedup 1.0000x reference)
import jax
import jax.numpy as jnp
from jax import lax
from jax.experimental import pallas as pl
from jax.experimental.pallas import tpu as pltpu
from jax.experimental.pallas import tpu_sc as plsc

N_FIELDS = 26
KEYS_PER_FIELD = 16
BATCH = 16384
LANES = 16
NUM_CORES = 2
NUM_SUBCORES = 16
NUM_WORKERS = NUM_CORES * NUM_SUBCORES
BCOLS = BATCH // NUM_WORKERS
CVECS = BCOLS // LANES


def _body(xt_hbm, out_hbm, x_v, out_v):
    wid = lax.axis_index("s") * NUM_CORES + lax.axis_index("c")
    base = wid * BCOLS
    pltpu.sync_copy(xt_hbm.at[:, pl.ds(base, BCOLS)], x_v)
    minus1 = jnp.full((LANES,), -1, dtype=jnp.int32)

    def group(g, carry):
        c0 = g * LANES
        for i in range(N_FIELDS):
            sl = pl.ds(c0, LANES)
            xv = x_v[i, sl]
            ok = xv.astype(jnp.uint32) < KEYS_PER_FIELD
            out_v[i, sl] = jnp.where(ok, xv + (i * KEYS_PER_FIELD), minus1)
        return carry

    lax.fori_loop(0, CVECS, group, 0)
    pltpu.sync_copy(out_v, out_hbm.at[:, pl.ds(base, BCOLS)])


@jax.jit
def kernel(x):
    run = pl.kernel(
        _body,
        out_type=jax.ShapeDtypeStruct((N_FIELDS, BATCH), jnp.int32),
        mesh=plsc.VectorSubcoreMesh(
            core_axis_name="c", subcore_axis_name="s",
            num_cores=NUM_CORES, num_subcores=NUM_SUBCORES,
        ),
        compiler_params=pltpu.CompilerParams(use_tc_tiling_on_sc=True),
        scratch_types=[
            pltpu.VMEM((N_FIELDS, BCOLS), jnp.int32),
            pltpu.VMEM((N_FIELDS, BCOLS), jnp.int32),
        ],
    )
    return run(x.T).T


# speedup vs baseline: 1.0647x; 1.0647x over previous
import jax
import jax.numpy as jnp
from jax import lax
from jax.experimental import pallas as pl
from jax.experimental.pallas import tpu as pltpu
from jax.experimental.pallas import tpu_sc as plsc

N_FIELDS = 26
KEYS_PER_FIELD = 16
BATCH = 16384
LANES = 16
NUM_CORES = 2
NUM_SUBCORES = 16
NUM_WORKERS = NUM_CORES * NUM_SUBCORES
BCOLS = BATCH // NUM_WORKERS
CVECS = BCOLS // LANES


def _body(xt_hbm, out_hbm, x_v, out_v):
    wid = lax.axis_index("s") * NUM_CORES + lax.axis_index("c")
    base = wid * BCOLS
    pltpu.sync_copy(xt_hbm.at[:, pl.ds(base, BCOLS)], x_v)
    minus1 = jnp.full((LANES,), -1, dtype=jnp.int32)

    @plsc.parallel_loop(0, CVECS)
    def group(g):
        c0 = g * LANES
        for i in range(N_FIELDS):
            sl = pl.ds(c0, LANES)
            xv = x_v[i, sl]
            ok = xv.astype(jnp.uint32) < KEYS_PER_FIELD
            out_v[i, sl] = jnp.where(ok, xv + (i * KEYS_PER_FIELD), minus1)
    pltpu.sync_copy(out_v, out_hbm.at[:, pl.ds(base, BCOLS)])


@jax.jit
def kernel(x):
    run = pl.kernel(
        _body,
        out_type=jax.ShapeDtypeStruct((N_FIELDS, BATCH), jnp.int32),
        mesh=plsc.VectorSubcoreMesh(
            core_axis_name="c", subcore_axis_name="s",
            num_cores=NUM_CORES, num_subcores=NUM_SUBCORES,
        ),
        compiler_params=pltpu.CompilerParams(use_tc_tiling_on_sc=True),
        scratch_types=[
            pltpu.VMEM((N_FIELDS, BCOLS), jnp.int32),
            pltpu.VMEM((N_FIELDS, BCOLS), jnp.int32),
        ],
    )
    return run(x.T).T
